# baseline (device time: 135520 ns/iter reference)
import jax
import jax.numpy as jnp
from jax import lax
from jax.experimental import pallas as pl
from jax.experimental.pallas import tpu as pltpu

N_DEV = 4
B, SQ, SKV = 2, 512, 512
H_PER = 8
DH = 64
D_MODEL = 768
D_SLICE = H_PER * DH
BLK = 64


def kernel(x, Wq, K_ext, V_ext, Wo):
    def body(x_ref, wq_ref, k_ref, v_ref, wo_ref, out_ref,
             comm_ref, ctx_ref, send_sems, recv_sems):
        my_pos = lax.axis_index("i")
        left = (my_pos - 1) % N_DEV
        right = (my_pos + 1) % N_DEV

        barrier_sem = pltpu.get_barrier_semaphore()
        for nbr in (left, right):
            pl.semaphore_signal(
                barrier_sem, inc=1,
                device_id=(nbr,), device_id_type=pl.DeviceIdType.MESH,
            )
        pl.semaphore_wait(barrier_sem, 2)

        qb4 = (lax.broadcasted_iota(jnp.int32, (SQ, SKV), 0) // BLK) % 4
        kb4 = (lax.broadcasted_iota(jnp.int32, (SQ, SKV), 1) // BLK) % 4
        mask = qb4 == kb4

        wq_slice = wq_ref[:, pl.ds(my_pos * D_SLICE, D_SLICE)]
        wo_slice = wo_ref[pl.ds(my_pos * D_SLICE, D_SLICE), :]

        for b in range(B):
            q_all = jnp.dot(x_ref[b], wq_slice,
                            preferred_element_type=jnp.float32)
            for h in range(H_PER):
                q = q_all[:, h * DH:(h + 1) * DH]
                k = k_ref[b, :, h, :]
                v = v_ref[b, :, h, :]
                s = lax.dot_general(
                    q, k, (((1,), (1,)), ((), ())),
                    preferred_element_type=jnp.float32) * 0.125
                s = jnp.where(mask, s, -1e9)
                m = jnp.max(s, axis=-1, keepdims=True)
                w = jnp.exp(s - m)
                w = w / jnp.sum(w, axis=-1, keepdims=True)
                ctx_ref[b, :, h * DH:(h + 1) * DH] = jnp.dot(
                    w, v, preferred_element_type=jnp.float32)
            partial = jnp.dot(ctx_ref[b], wo_slice,
                              preferred_element_type=jnp.float32)
            out_ref[b] = partial
            comm_ref[0, b] = partial

        for hop in range(N_DEV - 1):
            rdma = pltpu.make_async_remote_copy(
                src_ref=comm_ref.at[hop],
                dst_ref=comm_ref.at[hop + 1],
                send_sem=send_sems.at[hop],
                recv_sem=recv_sems.at[hop],
                device_id=(right,),
                device_id_type=pl.DeviceIdType.MESH,
            )
            rdma.start()
            rdma.wait()
            for b in range(B):
                out_ref[b] = out_ref[b] + comm_ref[hop + 1, b]

    return pl.pallas_call(
        body,
        out_shape=jax.ShapeDtypeStruct((B, SQ, D_MODEL), jnp.float32),
        in_specs=[pl.BlockSpec(memory_space=pltpu.VMEM)] * 5,
        out_specs=pl.BlockSpec(memory_space=pltpu.VMEM),
        scratch_shapes=[
            pltpu.VMEM((N_DEV, B, SQ, D_MODEL), jnp.float32),
            pltpu.VMEM((B, SQ, D_SLICE), jnp.float32),
            pltpu.SemaphoreType.DMA((N_DEV - 1,)),
            pltpu.SemaphoreType.DMA((N_DEV - 1,)),
        ],
        compiler_params=pltpu.CompilerParams(collective_id=0),
    )(x, Wq, K_ext, V_ext, Wo)


# device time: 60395 ns/iter; 2.2439x vs baseline; 2.2439x over previous
import jax
import jax.numpy as jnp
from jax import lax
from jax.experimental import pallas as pl
from jax.experimental.pallas import tpu as pltpu

N_DEV = 4
B, SQ, SKV = 2, 512, 512
H_PER = 8
DH = 64
D_MODEL = 768
D_SLICE = H_PER * DH
BLK = 64


def kernel(x, Wq, K_ext, V_ext, Wo):
    def body(x_ref, wq_ref, k_ref, v_ref, wo_ref, out_ref,
             ctx_ref, r1a, r1b, r2a, r2b, r3a, r3b, r4a, r4b,
             send_sems, recv_sems):
        my_pos = lax.axis_index("i")
        p1 = my_pos ^ 1
        p2 = 3 - my_pos

        barrier_sem = pltpu.get_barrier_semaphore()
        for nbr in (p1, p2):
            pl.semaphore_signal(
                barrier_sem, inc=1,
                device_id=(nbr,), device_id_type=pl.DeviceIdType.MESH,
            )
        pl.semaphore_wait(barrier_sem, 2)

        qb4 = (lax.broadcasted_iota(jnp.int32, (SQ, SKV), 0) // BLK) % 4
        kb4 = (lax.broadcasted_iota(jnp.int32, (SQ, SKV), 1) // BLK) % 4
        mask = qb4 == kb4

        wq_slice = wq_ref[:, pl.ds(my_pos * D_SLICE, D_SLICE)]
        wo_slice = wo_ref[pl.ds(my_pos * D_SLICE, D_SLICE), :]

        for b in range(B):
            q_all = jnp.dot(x_ref[b], wq_slice,
                            preferred_element_type=jnp.float32)
            for h in range(H_PER):
                q = q_all[:, h * DH:(h + 1) * DH]
                k = k_ref[b, :, h, :]
                v = v_ref[b, :, h, :]
                s = lax.dot_general(
                    q, k, (((1,), (1,)), ((), ())),
                    preferred_element_type=jnp.float32) * 0.125
                s = jnp.where(mask, s, -1e9)
                m = jnp.max(s, axis=-1, keepdims=True)
                w = jnp.exp(s - m)
                w = w / jnp.sum(w, axis=-1, keepdims=True)
                ctx_ref[b, :, h * DH:(h + 1) * DH] = jnp.dot(
                    w, v, preferred_element_type=jnp.float32)
            partial = jnp.dot(ctx_ref[b], wo_slice,
                              preferred_element_type=jnp.float32)
            out_ref[b] = partial

        in03 = jnp.logical_or(my_pos == 0, my_pos == 3)
        aS = jnp.where(in03, 0, 128)
        aSend = 128 - aS
        qA = aS + jnp.where(my_pos <= 1, 0, 64)
        qA_send = 2 * aS + 64 - qA
        bS = 256 + jnp.where(my_pos <= 1, 0, 128)
        bSend = 640 - bS
        qB = bS + jnp.where(my_pos % 2 == 0, 0, 64)
        qB_send = 2 * bS + 64 - qB

        def exchange(partner, send_start, length, rbuf, idx):
            rdma = pltpu.make_async_remote_copy(
                src_ref=out_ref.at[:, pl.ds(send_start, length), :],
                dst_ref=rbuf,
                send_sem=send_sems.at[idx],
                recv_sem=recv_sems.at[idx],
                device_id=(partner,),
                device_id_type=pl.DeviceIdType.MESH,
            )
            rdma.start()
            return rdma

        ra = exchange(p1, aSend, 128, r1a, 0)
        rb = exchange(p2, bSend, 128, r1b, 1)
        ra.wait()
        rb.wait()
        out_ref[:, pl.ds(aS, 128), :] = out_ref[:, pl.ds(aS, 128), :] + r1a[...]
        out_ref[:, pl.ds(bS, 128), :] = out_ref[:, pl.ds(bS, 128), :] + r1b[...]

        ra = exchange(p2, qA_send, 64, r2a, 2)
        rb = exchange(p1, qB_send, 64, r2b, 3)
        ra.wait()
        rb.wait()
        out_ref[:, pl.ds(qA, 64), :] = out_ref[:, pl.ds(qA, 64), :] + r2a[...]
        out_ref[:, pl.ds(qB, 64), :] = out_ref[:, pl.ds(qB, 64), :] + r2b[...]

        ra = exchange(p2, qA, 64, r3a, 4)
        rb = exchange(p1, qB, 64, r3b, 5)
        ra.wait()
        rb.wait()
        out_ref[:, pl.ds(qA_send, 64), :] = r3a[...]
        out_ref[:, pl.ds(qB_send, 64), :] = r3b[...]

        ra = exchange(p1, aS, 128, r4a, 6)
        rb = exchange(p2, bS, 128, r4b, 7)
        ra.wait()
        rb.wait()
        out_ref[:, pl.ds(aSend, 128), :] = r4a[...]
        out_ref[:, pl.ds(bSend, 128), :] = r4b[...]

    return pl.pallas_call(
        body,
        out_shape=jax.ShapeDtypeStruct((B, SQ, D_MODEL), jnp.float32),
        in_specs=[pl.BlockSpec(memory_space=pltpu.VMEM)] * 5,
        out_specs=pl.BlockSpec(memory_space=pltpu.VMEM),
        scratch_shapes=[
            pltpu.VMEM((B, SQ, D_SLICE), jnp.float32),
            pltpu.VMEM((B, 128, D_MODEL), jnp.float32),
            pltpu.VMEM((B, 128, D_MODEL), jnp.float32),
            pltpu.VMEM((B, 64, D_MODEL), jnp.float32),
            pltpu.VMEM((B, 64, D_MODEL), jnp.float32),
            pltpu.VMEM((B, 64, D_MODEL), jnp.float32),
            pltpu.VMEM((B, 64, D_MODEL), jnp.float32),
            pltpu.VMEM((B, 128, D_MODEL), jnp.float32),
            pltpu.VMEM((B, 128, D_MODEL), jnp.float32),
            pltpu.SemaphoreType.DMA((8,)),
            pltpu.SemaphoreType.DMA((8,)),
        ],
        compiler_params=pltpu.CompilerParams(collective_id=0),
    )(x, Wq, K_ext, V_ext, Wo)


# device time: 58205 ns/iter; 2.3283x vs baseline; 1.0376x over previous
import jax
import jax.numpy as jnp
from jax import lax
from jax.experimental import pallas as pl
from jax.experimental.pallas import tpu as pltpu

N_DEV = 4
B, SQ, SKV = 2, 512, 512
H_PER = 8
DH = 64
D_MODEL = 768
D_SLICE = H_PER * DH
BLK = 64


def kernel(x, Wq, K_ext, V_ext, Wo):
    def body(x_ref, wq_ref, k_ref, v_ref, wo_ref, out_ref,
             r1a, r1b, r2a, r2b, r3a, r3b, r4a, r4b,
             send_sems, recv_sems):
        my_pos = lax.axis_index("i")
        p1 = my_pos ^ 1
        p2 = 3 - my_pos

        barrier_sem = pltpu.get_barrier_semaphore()
        for nbr in (p1, p2):
            pl.semaphore_signal(
                barrier_sem, inc=1,
                device_id=(nbr,), device_id_type=pl.DeviceIdType.MESH,
            )
        pl.semaphore_wait(barrier_sem, 2)

        PERM = [0, 4, 1, 5, 2, 6, 3, 7]
        INV = [PERM.index(nb) for nb in range(8)]

        wq_slice = wq_ref[:, pl.ds(my_pos * D_SLICE, D_SLICE)]
        wo_slice = wo_ref[pl.ds(my_pos * D_SLICE, D_SLICE), :]

        for b in range(B):
            q_all = jnp.dot(x_ref[b], wq_slice,
                            preferred_element_type=jnp.float32)
            qp = jnp.concatenate(
                [q_all[blk * BLK:(blk + 1) * BLK] for blk in PERM], axis=0)
            k3 = k_ref[b]
            v3 = v_ref[b]
            kp = jnp.concatenate(
                [k3[blk * BLK:(blk + 1) * BLK] for blk in PERM], axis=0)
            vp = jnp.concatenate(
                [v3[blk * BLK:(blk + 1) * BLK] for blk in PERM], axis=0)
            cols = []
            for h in range(H_PER):
                pieces = []
                for g in range(4):
                    qs = qp[g * 128:(g + 1) * 128, h * DH:(h + 1) * DH]
                    ks = kp[g * 128:(g + 1) * 128, h, :]
                    vs = vp[g * 128:(g + 1) * 128, h, :]
                    s = lax.dot_general(
                        qs, ks, (((1,), (1,)), ((), ())),
                        preferred_element_type=jnp.float32) * 0.125
                    w = jnp.exp(s)
                    recip = 1.0 / jnp.sum(w, axis=-1, keepdims=True)
                    pieces.append(jnp.dot(
                        w, vs, preferred_element_type=jnp.float32) * recip)
                cols.append(jnp.concatenate(pieces, axis=0))
            ctxp = jnp.concatenate(cols, axis=1)
            ctx_nat = jnp.concatenate(
                [ctxp[INV[nb] * BLK:(INV[nb] + 1) * BLK] for nb in range(8)],
                axis=0)
            out_ref[b] = jnp.dot(ctx_nat, wo_slice,
                                 preferred_element_type=jnp.float32)

        in03 = jnp.logical_or(my_pos == 0, my_pos == 3)
        aS = jnp.where(in03, 0, 128)
        aSend = 128 - aS
        qA = aS + jnp.where(my_pos <= 1, 0, 64)
        qA_send = 2 * aS + 64 - qA
        bS = 256 + jnp.where(my_pos <= 1, 0, 128)
        bSend = 640 - bS
        qB = bS + jnp.where(my_pos % 2 == 0, 0, 64)
        qB_send = 2 * bS + 64 - qB

        def exchange(partner, send_start, length, rbuf, idx):
            rdma = pltpu.make_async_remote_copy(
                src_ref=out_ref.at[:, pl.ds(send_start, length), :],
                dst_ref=rbuf,
                send_sem=send_sems.at[idx],
                recv_sem=recv_sems.at[idx],
                device_id=(partner,),
                device_id_type=pl.DeviceIdType.MESH,
            )
            rdma.start()
            return rdma

        ra = exchange(p1, aSend, 128, r1a, 0)
        rb = exchange(p2, bSend, 128, r1b, 1)
        ra.wait()
        rb.wait()
        out_ref[:, pl.ds(aS, 128), :] = out_ref[:, pl.ds(aS, 128), :] + r1a[...]
        out_ref[:, pl.ds(bS, 128), :] = out_ref[:, pl.ds(bS, 128), :] + r1b[...]

        ra = exchange(p2, qA_send, 64, r2a, 2)
        rb = exchange(p1, qB_send, 64, r2b, 3)
        ra.wait()
        rb.wait()
        out_ref[:, pl.ds(qA, 64), :] = out_ref[:, pl.ds(qA, 64), :] + r2a[...]
        out_ref[:, pl.ds(qB, 64), :] = out_ref[:, pl.ds(qB, 64), :] + r2b[...]

        ra = exchange(p2, qA, 64, r3a, 4)
        rb = exchange(p1, qB, 64, r3b, 5)
        ra.wait()
        rb.wait()
        out_ref[:, pl.ds(qA_send, 64), :] = r3a[...]
        out_ref[:, pl.ds(qB_send, 64), :] = r3b[...]

        ra = exchange(p1, aS, 128, r4a, 6)
        rb = exchange(p2, bS, 128, r4b, 7)
        ra.wait()
        rb.wait()
        out_ref[:, pl.ds(aSend, 128), :] = r4a[...]
        out_ref[:, pl.ds(bSend, 128), :] = r4b[...]

    return pl.pallas_call(
        body,
        out_shape=jax.ShapeDtypeStruct((B, SQ, D_MODEL), jnp.float32),
        in_specs=[pl.BlockSpec(memory_space=pltpu.VMEM)] * 5,
        out_specs=pl.BlockSpec(memory_space=pltpu.VMEM),
        scratch_shapes=[
            pltpu.VMEM((B, 128, D_MODEL), jnp.float32),
            pltpu.VMEM((B, 128, D_MODEL), jnp.float32),
            pltpu.VMEM((B, 64, D_MODEL), jnp.float32),
            pltpu.VMEM((B, 64, D_MODEL), jnp.float32),
            pltpu.VMEM((B, 64, D_MODEL), jnp.float32),
            pltpu.VMEM((B, 64, D_MODEL), jnp.float32),
            pltpu.VMEM((B, 128, D_MODEL), jnp.float32),
            pltpu.VMEM((B, 128, D_MODEL), jnp.float32),
            pltpu.SemaphoreType.DMA((8,)),
            pltpu.SemaphoreType.DMA((8,)),
        ],
        compiler_params=pltpu.CompilerParams(collective_id=0),
    )(x, Wq, K_ext, V_ext, Wo)
